# baseline (speedup 1.0000x reference)
import functools

import jax
import jax.numpy as jnp
from jax import lax
from jax.experimental import pallas as pl
from jax.experimental.pallas import tpu as pltpu
from jax.experimental.pallas import tpu_sc as plsc

_B = 4096
_L = 200
_D = 64
_HIDDEN = 256
_NCLS = 2

_NC = 2
_NS = 16
_NW = _NC * _NS
_RPW = _B // _NW
_IPW = _RPW * _L
_NBUF = 2


def _sc_pool(ids_w, ids_b, ids_t, emb_w, emb_b, emb_t):
    mesh = plsc.VectorSubcoreMesh(
        core_axis_name="c", subcore_axis_name="s",
        num_cores=_NC, num_subcores=_NS)

    out_type = (
        jax.ShapeDtypeStruct((_B, _D), jnp.float32),
        jax.ShapeDtypeStruct((_B, _D), jnp.float32),
        jax.ShapeDtypeStruct((_B, _D), jnp.float32),
    )

    scratch = [
        pltpu.VMEM((_RPW, _L), jnp.int32),
        pltpu.VMEM((_RPW, _L), jnp.int32),
        pltpu.VMEM((_L, _D), jnp.float32),
        pltpu.VMEM((_L, _D), jnp.float32),
        pltpu.VMEM((_RPW, _D), jnp.float32),
        pltpu.SemaphoreType.DMA,
        pltpu.SemaphoreType.DMA,
        pltpu.VMEM_SHARED((_NS * _RPW, _D), jnp.float32),
        pltpu.VMEM_SHARED((_NS * _RPW, _D), jnp.float32),
        pltpu.VMEM_SHARED((_NS * _RPW, _D), jnp.float32),
    ]

    @functools.partial(pl.kernel, mesh=mesh, out_type=out_type,
                       scratch_types=scratch,
                       compiler_params=pltpu.CompilerParams(
                           use_tc_tiling_on_sc=False))
    def k(ids_w_h, ids_b_h, ids_t_h, emb_w_h, emb_b_h, emb_t_h,
          out_w_h, out_b_h, out_t_h,
          idx_v, bag_v, rows_v0, rows_v1, tmp_v, sem0, sem1,
          acc_w, acc_b, acc_t):
        rows = (rows_v0, rows_v1)
        sems = (sem0, sem1)
        cid = lax.axis_index("c")
        sid = lax.axis_index("s")
        wid = cid * _NS + sid
        sbase = pl.multiple_of(sid * _RPW, _RPW)
        gbase = pl.multiple_of(wid * _RPW, _RPW)

        lanes = lax.iota(jnp.int32, 16)

        @pl.loop(0, _RPW)
        def _(r):
            val = lanes * 0 + (sbase + r)
            for i in range(_L // 16):
                bag_v[r, pl.ds(i * 16, 16)] = val
            bag_v[r, pl.ds(_L - 16, 16)] = val
            for o in range(0, _D, 16):
                tmp_v[r, pl.ds(o, 16)] = jnp.float32(0.0) * lanes

        for acc in (acc_w, acc_b, acc_t):
            pltpu.sync_copy(tmp_v, acc.at[pl.ds(sbase, _RPW)])

        for ids_h, emb_h, acc in ((ids_w_h, emb_w_h, acc_w),
                                  (ids_b_h, emb_b_h, acc_b),
                                  (ids_t_h, emb_t_h, acc_t)):
            pltpu.sync_copy(ids_h.at[pl.ds(gbase, _RPW)], idx_v)

            def gcopy(r, b, _emb_h=emb_h):
                return pltpu.make_async_copy(
                    _emb_h.at[idx_v.at[r]], rows[b], sems[b])

            for b in range(_NBUF):
                gcopy(b, b).start()

            @pl.loop(0, _RPW - _NBUF, step=_NBUF)
            def _(r0):
                for b in range(_NBUF):
                    r = r0 + b
                    gcopy(0, b).wait()
                    pltpu.sync_copy(rows[b], acc.at[bag_v.at[r]], add=True)
                    gcopy(r + _NBUF, b).start()

            for b in range(_NBUF):
                r = _RPW - _NBUF + b
                gcopy(0, b).wait()
                pltpu.sync_copy(rows[b], acc.at[bag_v.at[r]], add=True)

        for acc, out_h in ((acc_w, out_w_h), (acc_b, out_b_h),
                           (acc_t, out_t_h)):
            pltpu.sync_copy(acc.at[pl.ds(sbase, _RPW)], tmp_v)
            pltpu.sync_copy(tmp_v, out_h.at[pl.ds(gbase, _RPW)])

    return k(ids_w, ids_b, ids_t, emb_w, emb_b, emb_t)


def _mlp_body(xw_ref, xb_ref, xt_ref, w1_ref, b1_ref, w2_ref, b2_ref,
              out_ref):
    scale = jnp.float32(1.0 / _L)
    h = jnp.dot(xw_ref[...], w1_ref[0:_D, :],
                preferred_element_type=jnp.float32)
    h += jnp.dot(xb_ref[...], w1_ref[_D:2 * _D, :],
                 preferred_element_type=jnp.float32)
    h += jnp.dot(xt_ref[...], w1_ref[2 * _D:3 * _D, :],
                 preferred_element_type=jnp.float32)
    h = h * scale + b1_ref[...]
    h = jnp.maximum(h, 0.0)
    out_ref[...] = jnp.dot(h, w2_ref[...],
                           preferred_element_type=jnp.float32) + b2_ref[...]


def _mlp(xw, xb, xt, w1, b1, w2p, b2p):
    blk = 512
    grid = (_B // blk,)
    return pl.pallas_call(
        _mlp_body,
        grid=grid,
        in_specs=[
            pl.BlockSpec((blk, _D), lambda i: (i, 0)),
            pl.BlockSpec((blk, _D), lambda i: (i, 0)),
            pl.BlockSpec((blk, _D), lambda i: (i, 0)),
            pl.BlockSpec((3 * _D, _HIDDEN), lambda i: (0, 0)),
            pl.BlockSpec((1, _HIDDEN), lambda i: (0, 0)),
            pl.BlockSpec((_HIDDEN, 128), lambda i: (0, 0)),
            pl.BlockSpec((1, 128), lambda i: (0, 0)),
        ],
        out_specs=pl.BlockSpec((blk, 128), lambda i: (i, 0)),
        out_shape=jax.ShapeDtypeStruct((_B, 128), jnp.float32),
    )(xw, xb, xt, w1, b1, w2p, b2p)


def kernel(input_ids, bigram, trigram, seq_len, emb_word, emb_bi, emb_tri,
           W1, b1, W2, b2):
    del seq_len
    xw, xb, xt = _sc_pool(input_ids, bigram, trigram,
                          emb_word, emb_bi, emb_tri)

    w2p = jnp.zeros((_HIDDEN, 128), jnp.float32).at[:, :_NCLS].set(W2)
    b2p = jnp.zeros((1, 128), jnp.float32).at[0, :_NCLS].set(b2)
    out = _mlp(xw, xb, xt, W1, b1.reshape(1, _HIDDEN), w2p, b2p)
    return out[:, :_NCLS]


# speedup vs baseline: 1.0024x; 1.0024x over previous
import functools

import jax
import jax.numpy as jnp
from jax import lax
from jax.experimental import pallas as pl
from jax.experimental.pallas import tpu as pltpu
from jax.experimental.pallas import tpu_sc as plsc

_B = 4096
_L = 200
_D = 64
_HIDDEN = 256
_NCLS = 2

_NC = 2
_NS = 16
_NW = _NC * _NS
_RPW = _B // _NW
_IPW = _RPW * _L
_NBUF = 2


def _sc_pool(ids_w, ids_b, ids_t, emb_w, emb_b, emb_t):
    mesh = plsc.VectorSubcoreMesh(
        core_axis_name="c", subcore_axis_name="s",
        num_cores=_NC, num_subcores=_NS)

    out_type = (
        jax.ShapeDtypeStruct((_B, _D), jnp.float32),
        jax.ShapeDtypeStruct((_B, _D), jnp.float32),
        jax.ShapeDtypeStruct((_B, _D), jnp.float32),
    )

    scratch = [
        pltpu.VMEM((_IPW,), jnp.int32),
        pltpu.VMEM((_RPW, _L), jnp.int32),
        pltpu.VMEM((_L, _D), jnp.float32),
        pltpu.VMEM((_L, _D), jnp.float32),
        pltpu.VMEM((_RPW, _D), jnp.float32),
        pltpu.SemaphoreType.DMA,
        pltpu.SemaphoreType.DMA,
        pltpu.VMEM_SHARED((_NS * _RPW, _D), jnp.float32),
        pltpu.VMEM_SHARED((_NS * _RPW, _D), jnp.float32),
        pltpu.VMEM_SHARED((_NS * _RPW, _D), jnp.float32),
    ]

    @functools.partial(pl.kernel, mesh=mesh, out_type=out_type,
                       scratch_types=scratch,
                       compiler_params=pltpu.CompilerParams(
                           use_tc_tiling_on_sc=False))
    def k(ids_w_h, ids_b_h, ids_t_h, emb_w_h, emb_b_h, emb_t_h,
          out_w_h, out_b_h, out_t_h,
          idx_v, bag_v, rows_v0, rows_v1, tmp_v, sem0, sem1,
          acc_w, acc_b, acc_t):
        rows = (rows_v0, rows_v1)
        sems = (sem0, sem1)
        cid = lax.axis_index("c")
        sid = lax.axis_index("s")
        wid = cid * _NS + sid
        sbase = pl.multiple_of(sid * _RPW, _RPW)
        gbase = pl.multiple_of(wid * _RPW, _RPW)
        ibase = pl.multiple_of(wid * _IPW, _IPW)

        lanes = lax.iota(jnp.int32, 16)

        @pl.loop(0, _RPW)
        def _(r):
            val = lanes * 0 + (sbase + r)
            for i in range(_L // 16):
                bag_v[r, pl.ds(i * 16, 16)] = val
            bag_v[r, pl.ds(_L - 16, 16)] = val
            for o in range(0, _D, 16):
                tmp_v[r, pl.ds(o, 16)] = jnp.float32(0.0) * lanes

        for acc in (acc_w, acc_b, acc_t):
            pltpu.sync_copy(tmp_v, acc.at[pl.ds(sbase, _RPW)])

        for ids_h, emb_h, acc in ((ids_w_h, emb_w_h, acc_w),
                                  (ids_b_h, emb_b_h, acc_b),
                                  (ids_t_h, emb_t_h, acc_t)):
            pltpu.sync_copy(ids_h.at[pl.ds(ibase, _IPW)], idx_v)

            def gcopy(r, b, _emb_h=emb_h):
                off = pl.multiple_of(r * _L, 8)
                return pltpu.make_async_copy(
                    _emb_h.at[idx_v.at[pl.ds(off, _L)]], rows[b], sems[b])

            for b in range(_NBUF):
                gcopy(b, b).start()

            @pl.loop(0, _RPW - _NBUF, step=_NBUF)
            def _(r0):
                for b in range(_NBUF):
                    r = r0 + b
                    gcopy(0, b).wait()
                    pltpu.sync_copy(rows[b], acc.at[bag_v.at[r]], add=True)
                    gcopy(r + _NBUF, b).start()

            for b in range(_NBUF):
                r = _RPW - _NBUF + b
                gcopy(0, b).wait()
                pltpu.sync_copy(rows[b], acc.at[bag_v.at[r]], add=True)

        for acc, out_h in ((acc_w, out_w_h), (acc_b, out_b_h),
                           (acc_t, out_t_h)):
            pltpu.sync_copy(acc.at[pl.ds(sbase, _RPW)],
                            out_h.at[pl.ds(gbase, _RPW)])

    return k(ids_w, ids_b, ids_t, emb_w, emb_b, emb_t)


def _mlp_body(xw_ref, xb_ref, xt_ref, w1_ref, b1_ref, w2_ref, b2_ref,
              out_ref):
    scale = jnp.float32(1.0 / _L)
    h = jnp.dot(xw_ref[...], w1_ref[0:_D, :],
                preferred_element_type=jnp.float32)
    h += jnp.dot(xb_ref[...], w1_ref[_D:2 * _D, :],
                 preferred_element_type=jnp.float32)
    h += jnp.dot(xt_ref[...], w1_ref[2 * _D:3 * _D, :],
                 preferred_element_type=jnp.float32)
    h = h * scale + b1_ref[...]
    h = jnp.maximum(h, 0.0)
    out_ref[...] = jnp.dot(h, w2_ref[...],
                           preferred_element_type=jnp.float32) + b2_ref[...]


def _mlp(xw, xb, xt, w1, b1, w2p, b2p):
    blk = 512
    grid = (_B // blk,)
    return pl.pallas_call(
        _mlp_body,
        grid=grid,
        in_specs=[
            pl.BlockSpec((blk, _D), lambda i: (i, 0)),
            pl.BlockSpec((blk, _D), lambda i: (i, 0)),
            pl.BlockSpec((blk, _D), lambda i: (i, 0)),
            pl.BlockSpec((3 * _D, _HIDDEN), lambda i: (0, 0)),
            pl.BlockSpec((1, _HIDDEN), lambda i: (0, 0)),
            pl.BlockSpec((_HIDDEN, 128), lambda i: (0, 0)),
            pl.BlockSpec((1, 128), lambda i: (0, 0)),
        ],
        out_specs=pl.BlockSpec((blk, 128), lambda i: (i, 0)),
        out_shape=jax.ShapeDtypeStruct((_B, 128), jnp.float32),
    )(xw, xb, xt, w1, b1, w2p, b2p)


def kernel(input_ids, bigram, trigram, seq_len, emb_word, emb_bi, emb_tri,
           W1, b1, W2, b2):
    del seq_len
    xw, xb, xt = _sc_pool(input_ids.reshape(-1), bigram.reshape(-1),
                          trigram.reshape(-1), emb_word, emb_bi, emb_tri)

    w2p = jnp.zeros((_HIDDEN, 128), jnp.float32).at[:, :_NCLS].set(W2)
    b2p = jnp.zeros((1, 128), jnp.float32).at[0, :_NCLS].set(b2)
    out = _mlp(xw, xb, xt, W1, b1.reshape(1, _HIDDEN), w2p, b2p)
    return out[:, :_NCLS]


# speedup vs baseline: 1.0180x; 1.0155x over previous
import functools

import jax
import jax.numpy as jnp
from jax import lax
from jax.experimental import pallas as pl
from jax.experimental.pallas import tpu as pltpu
from jax.experimental.pallas import tpu_sc as plsc

_B = 4096
_L = 200
_D = 64
_HIDDEN = 256
_NCLS = 2

_NC = 2
_NS = 16
_NW = _NC * _NS
_RPW = _B // _NW
_IPW = _RPW * _L
_NBUF = 2
_C = 2 * _L
_NCH = _IPW // _C


def _sc_pool(ids_w, ids_b, ids_t, emb_w, emb_b, emb_t):
    mesh = plsc.VectorSubcoreMesh(
        core_axis_name="c", subcore_axis_name="s",
        num_cores=_NC, num_subcores=_NS)

    out_type = (
        jax.ShapeDtypeStruct((_B, _D), jnp.float32),
        jax.ShapeDtypeStruct((_B, _D), jnp.float32),
        jax.ShapeDtypeStruct((_B, _D), jnp.float32),
    )

    scratch = [
        pltpu.VMEM((_IPW,), jnp.int32),
        pltpu.VMEM((_IPW,), jnp.int32),
        pltpu.VMEM((_C, _D), jnp.float32),
        pltpu.VMEM((_C, _D), jnp.float32),
        pltpu.VMEM((32, _D), jnp.float32),
        pltpu.SemaphoreType.DMA,
        pltpu.SemaphoreType.DMA,
        pltpu.VMEM_SHARED((_NS * _RPW, _D), jnp.float32),
        pltpu.VMEM_SHARED((_NS * _RPW, _D), jnp.float32),
        pltpu.VMEM_SHARED((_NS * _RPW, _D), jnp.float32),
    ]

    @functools.partial(pl.kernel, mesh=mesh, out_type=out_type,
                       scratch_types=scratch,
                       compiler_params=pltpu.CompilerParams(
                           use_tc_tiling_on_sc=False))
    def k(ids_w_h, ids_b_h, ids_t_h, emb_w_h, emb_b_h, emb_t_h,
          out_w_h, out_b_h, out_t_h,
          idx_v, bag_v, rows_v0, rows_v1, tmp_v, sem0, sem1,
          acc_w, acc_b, acc_t):
        rows = (rows_v0, rows_v1)
        sems = (sem0, sem1)
        cid = lax.axis_index("c")
        sid = lax.axis_index("s")
        wid = cid * _NS + sid
        sbase = pl.multiple_of(sid * _RPW, _RPW)
        gbase = pl.multiple_of(wid * _RPW, _RPW)
        ibase = pl.multiple_of(wid * _IPW, _IPW)

        lanes = lax.iota(jnp.int32, 16)
        blocks = [jnp.where(lanes + 16 * j >= _L, 1, 0).astype(jnp.int32)
                  for j in range(_C // 16)]

        @pl.loop(0, _RPW // 2)
        def _(p):
            off = pl.multiple_of(p * _C, 16)
            v0 = sbase + p * 2
            for j in range(_C // 16):
                bag_v[pl.ds(off + 16 * j, 16)] = blocks[j] + v0

        @pl.loop(0, 32)
        def _(r):
            for o in range(0, _D, 16):
                tmp_v[r, pl.ds(o, 16)] = jnp.float32(0.0) * lanes

        for acc in (acc_w, acc_b, acc_t):
            for r0 in range(0, _RPW, 32):
                pltpu.sync_copy(tmp_v, acc.at[pl.ds(sbase + r0, 32)])

        for ids_h, emb_h, acc in ((ids_w_h, emb_w_h, acc_w),
                                  (ids_b_h, emb_b_h, acc_b),
                                  (ids_t_h, emb_t_h, acc_t)):
            pltpu.sync_copy(ids_h.at[pl.ds(ibase, _IPW)], idx_v)

            def gcopy(c, b, _emb_h=emb_h):
                off = pl.multiple_of(c * _C, 16)
                return pltpu.make_async_copy(
                    _emb_h.at[idx_v.at[pl.ds(off, _C)]], rows[b], sems[b])

            def scatter(c, b, _acc=acc):
                off = pl.multiple_of(c * _C, 16)
                pltpu.sync_copy(rows[b], _acc.at[bag_v.at[pl.ds(off, _C)]],
                                add=True)

            for b in range(_NBUF):
                gcopy(b, b).start()

            @pl.loop(0, _NCH - _NBUF, step=_NBUF)
            def _(c0):
                for b in range(_NBUF):
                    c = c0 + b
                    gcopy(0, b).wait()
                    scatter(c, b)
                    gcopy(c + _NBUF, b).start()

            for b in range(_NBUF):
                c = _NCH - _NBUF + b
                gcopy(0, b).wait()
                scatter(c, b)

        for acc, out_h in ((acc_w, out_w_h), (acc_b, out_b_h),
                           (acc_t, out_t_h)):
            pltpu.sync_copy(acc.at[pl.ds(sbase, _RPW)],
                            out_h.at[pl.ds(gbase, _RPW)])

    return k(ids_w, ids_b, ids_t, emb_w, emb_b, emb_t)


def _mlp_body(xw_ref, xb_ref, xt_ref, w1_ref, b1_ref, w2_ref, b2_ref,
              out_ref):
    scale = jnp.float32(1.0 / _L)
    h = jnp.dot(xw_ref[...], w1_ref[0:_D, :],
                preferred_element_type=jnp.float32)
    h += jnp.dot(xb_ref[...], w1_ref[_D:2 * _D, :],
                 preferred_element_type=jnp.float32)
    h += jnp.dot(xt_ref[...], w1_ref[2 * _D:3 * _D, :],
                 preferred_element_type=jnp.float32)
    h = h * scale + b1_ref[...]
    h = jnp.maximum(h, 0.0)
    out_ref[...] = jnp.dot(h, w2_ref[...],
                           preferred_element_type=jnp.float32) + b2_ref[...]


def _mlp(xw, xb, xt, w1, b1, w2p, b2p):
    blk = 512
    grid = (_B // blk,)
    return pl.pallas_call(
        _mlp_body,
        grid=grid,
        in_specs=[
            pl.BlockSpec((blk, _D), lambda i: (i, 0)),
            pl.BlockSpec((blk, _D), lambda i: (i, 0)),
            pl.BlockSpec((blk, _D), lambda i: (i, 0)),
            pl.BlockSpec((3 * _D, _HIDDEN), lambda i: (0, 0)),
            pl.BlockSpec((1, _HIDDEN), lambda i: (0, 0)),
            pl.BlockSpec((_HIDDEN, 128), lambda i: (0, 0)),
            pl.BlockSpec((1, 128), lambda i: (0, 0)),
        ],
        out_specs=pl.BlockSpec((blk, 128), lambda i: (i, 0)),
        out_shape=jax.ShapeDtypeStruct((_B, 128), jnp.float32),
    )(xw, xb, xt, w1, b1, w2p, b2p)


def kernel(input_ids, bigram, trigram, seq_len, emb_word, emb_bi, emb_tri,
           W1, b1, W2, b2):
    del seq_len
    xw, xb, xt = _sc_pool(input_ids.reshape(-1), bigram.reshape(-1),
                          trigram.reshape(-1), emb_word, emb_bi, emb_tri)

    w2p = jnp.zeros((_HIDDEN, 128), jnp.float32).at[:, :_NCLS].set(W2)
    b2p = jnp.zeros((1, 128), jnp.float32).at[0, :_NCLS].set(b2)
    out = _mlp(xw, xb, xt, W1, b1.reshape(1, _HIDDEN), w2p, b2p)
    return out[:, :_NCLS]
